# Initial kernel scaffold; baseline (speedup 1.0000x reference)
import jax
import jax.numpy as jnp
from jax.experimental import pallas as pl


def kernel(x, edge_index0, edge_index1, pos_edge_index, neg_edge_index, W_self1, W_neigh1, b1, W_self2, W_neigh2, b2, Wp1, bp1, Wp2, bp2):
    raise NotImplementedError("write your pallas kernel here")


# speedup vs baseline: 1.5389x; 1.5389x over previous
import functools

import jax
import jax.numpy as jnp
from jax import lax
from jax.experimental import pallas as pl
from jax.experimental.pallas import tpu as pltpu
from jax.experimental.pallas import tpu_sc as plsc

_N = 10000
_NP = 10240
_E = 160000
_EP = 163840
_NC = 2
_NS = 16
_P = 53248


def _tc1(x, w_self1, w_neigh1):
    R = 1000

    def body(x_ref, ws_ref, wn_ref, xs_ref, xw_ref):
        xx = x_ref[...]
        xs_ref[...] = jnp.dot(xx, ws_ref[...], preferred_element_type=jnp.float32)
        xw = jnp.dot(xx, wn_ref[...], preferred_element_type=jnp.float32)
        xw_ref[0, :, :] = xw[:, :128]
        xw_ref[1, :, :] = xw[:, 128:]

    return pl.pallas_call(
        body,
        grid=(_N // R,),
        in_specs=[pl.BlockSpec((R, 512), lambda i: (i, 0)),
                  pl.BlockSpec((512, 256), lambda i: (0, 0)),
                  pl.BlockSpec((512, 256), lambda i: (0, 0))],
        out_specs=[pl.BlockSpec((R, 256), lambda i: (i, 0)),
                   pl.BlockSpec((2, R, 128), lambda i: (0, i, 0))],
        out_shape=[jax.ShapeDtypeStruct((_N, 256), jnp.float32),
                   jax.ShapeDtypeStruct((2, _N, 128), jnp.float32)],
    )(x, w_self1, w_neigh1)


def _tc2(xs, seg1, deg1, b1, w_neigh2, w_self2):
    R = 1000

    def body(xs_ref, sg_ref, dg_ref, b1_ref, wn_ref, ws_ref, hw_ref, hs_ref):
        deg = jnp.maximum(dg_ref[:, 0:1], 1.0)
        seg = jnp.concatenate([sg_ref[0], sg_ref[1]], axis=-1)
        h = jnp.maximum(xs_ref[...] + seg / deg + b1_ref[...], 0.0)
        hw = jnp.dot(h, wn_ref[...], preferred_element_type=jnp.float32)
        hw_ref[...] = jnp.concatenate(
            [hw, jnp.ones((R, 64), jnp.float32)], axis=1)
        hs_ref[...] = jnp.dot(h, ws_ref[...], preferred_element_type=jnp.float32)

    return pl.pallas_call(
        body,
        grid=(_N // R,),
        in_specs=[pl.BlockSpec((R, 256), lambda i: (i, 0)),
                  pl.BlockSpec((2, R, 128), lambda i: (0, i, 0)),
                  pl.BlockSpec((R, 128), lambda i: (i, 0)),
                  pl.BlockSpec((1, 256), lambda i: (0, 0)),
                  pl.BlockSpec((256, 64), lambda i: (0, 0)),
                  pl.BlockSpec((256, 64), lambda i: (0, 0))],
        out_specs=[pl.BlockSpec((R, 128), lambda i: (i, 0)),
                   pl.BlockSpec((R, 64), lambda i: (i, 0))],
        out_shape=[jax.ShapeDtypeStruct((_N, 128), jnp.float32),
                   jax.ShapeDtypeStruct((_N, 64), jnp.float32)],
    )(xs, seg1, deg1, b1, w_neigh2, w_self2)


def _tc3(hs2, seg2, b2, wp1a, wp1b, bp1):
    R = 1000

    def body(hs_ref, sg_ref, b2_ref, wa_ref, wb_ref, bp_ref, ab_ref):
        seg = sg_ref[0] + sg_ref[1]
        deg = jnp.maximum(seg[:, 64:65], 1.0)
        h2 = hs_ref[...] + seg[:, :64] / deg + b2_ref[...]
        a = jnp.dot(h2, wa_ref[...], preferred_element_type=jnp.float32) + bp_ref[...]
        b = jnp.dot(h2, wb_ref[...], preferred_element_type=jnp.float32)
        ab_ref[...] = jnp.concatenate([a, b], axis=1)

    return pl.pallas_call(
        body,
        grid=(_N // R,),
        in_specs=[pl.BlockSpec((R, 64), lambda i: (i, 0)),
                  pl.BlockSpec((2, R, 128), lambda i: (0, i, 0)),
                  pl.BlockSpec((1, 64), lambda i: (0, 0)),
                  pl.BlockSpec((64, 64), lambda i: (0, 0)),
                  pl.BlockSpec((64, 64), lambda i: (0, 0)),
                  pl.BlockSpec((1, 64), lambda i: (0, 0))],
        out_specs=[pl.BlockSpec((R, 128), lambda i: (i, 0))],
        out_shape=[jax.ShapeDtypeStruct((_N, 128), jnp.float32)],
    )(hs2, seg2, b2, wp1a, wp1b, bp1)[0]


def _tc4(g, w2row, bp2):
    R = 3328

    def body(g_ref, w2_ref, bp2_ref, pos_ref, neg_ref):
        zp = jnp.maximum(g_ref[0, :, :64] + g_ref[1, :, 64:], 0.0)
        zn = jnp.maximum(g_ref[2, :, :64] + g_ref[3, :, 64:], 0.0)
        pos_ref[...] = jnp.sum(zp * w2_ref[...], axis=1, keepdims=True) + bp2_ref[...]
        neg_ref[...] = jnp.sum(zn * w2_ref[...], axis=1, keepdims=True) + bp2_ref[...]

    return pl.pallas_call(
        body,
        grid=(_P // R,),
        in_specs=[pl.BlockSpec((4, R, 128), lambda i: (0, i, 0)),
                  pl.BlockSpec((1, 64), lambda i: (0, 0)),
                  pl.BlockSpec((1, 1), lambda i: (0, 0))],
        out_specs=[pl.BlockSpec((R, 1), lambda i: (i, 0)),
                   pl.BlockSpec((R, 1), lambda i: (i, 0))],
        out_shape=[jax.ShapeDtypeStruct((_P, 1), jnp.float32),
                   jax.ShapeDtypeStruct((_P, 1), jnp.float32)],
    )(g, w2row, bp2)


_MESH = plsc.VectorSubcoreMesh(core_axis_name="c", subcore_axis_name="s")

_CH = 128
_HB = 64
_RW = _NP // (_NS * _HB)


def _split64(src128_v, dst64_v, h):
    for q in range(4):
        dst64_v[pl.ds(q * 16, 16)] = src128_v[pl.ds(h * 64 + q * 16, 16)]


def _sc_seg1(xw, src0, dst0, z128, ones128):
    NQ = _EP // _CH
    NCH = NQ // _NS

    @functools.partial(
        pl.kernel,
        out_type=(jax.ShapeDtypeStruct((2, _NP, 128), jnp.float32),
                  jax.ShapeDtypeStruct((2, _NP, 128), jnp.float32)),
        mesh=_MESH,
        scratch_types=[
            pltpu.VMEM_SHARED((_NP, 128), jnp.float32),
            pltpu.VMEM((_CH,), jnp.int32),
            pltpu.VMEM((_CH,), jnp.int32),
            pltpu.VMEM((_HB,), jnp.int32),
            pltpu.VMEM((_HB,), jnp.int32),
            pltpu.VMEM((_HB, 128), jnp.float32),
        ],
    )
    def k(xw_hbm, src0_hbm, dst0_hbm, z128_hbm, ones_hbm,
          seg_hbm, deg_hbm,
          acc_sh, src_v, dst_v, idx64_v, dst64_v, rows_v):
        c = lax.axis_index("c")
        s = lax.axis_index("s")

        def zero_acc():
            pltpu.sync_copy(z128_hbm, rows_v)
            for j in range(_RW):
                off = (s * _RW + j) * _HB
                pltpu.sync_copy(rows_v, acc_sh.at[pl.ds(off, _HB)])

        def write_acc(out3_hbm):
            for j in range(_RW):
                off = (s * _RW + j) * _HB
                pltpu.sync_copy(acc_sh.at[pl.ds(off, _HB)], rows_v)
                pltpu.sync_copy(rows_v, out3_hbm.at[c].at[pl.ds(off, _HB)])

        zero_acc()
        plsc.subcore_barrier()

        def body(kk, carry):
            off = (s + kk * _NS) * _CH
            pltpu.sync_copy(src0_hbm.at[pl.ds(off, _CH)], src_v)
            pltpu.sync_copy(dst0_hbm.at[pl.ds(off, _CH)], dst_v)
            for h in range(2):
                _split64(src_v, idx64_v, h)
                _split64(dst_v, dst64_v, h)
                pltpu.sync_copy(xw_hbm.at[c].at[idx64_v], rows_v)
                pltpu.sync_copy(rows_v, acc_sh.at[dst64_v], add=True)
            return carry

        lax.fori_loop(0, NCH, body, 0)
        plsc.subcore_barrier()
        write_acc(seg_hbm)
        zero_acc()
        plsc.subcore_barrier()

        pltpu.sync_copy(ones_hbm, rows_v)

        def body2(kk, carry):
            off = (s + kk * _NS) * _CH
            pltpu.sync_copy(dst0_hbm.at[pl.ds(off, _CH)], dst_v)
            for h in range(2):
                _split64(dst_v, dst64_v, h)
                pltpu.sync_copy(rows_v, acc_sh.at[dst64_v], add=True)
            return carry

        lax.fori_loop(0, NCH, body2, 0)
        plsc.subcore_barrier()
        write_acc(deg_hbm)

    return k(xw, src0, dst0, z128, ones128)


_RN = 2560
_NR = 4
_TRASH = _RN


def _sc_seg2(hw2, src, dst, z128):
    NQ = _EP // _CH
    NW = _NC * _NS
    NCH = NQ // NW
    AR = 2688
    WR = _RN // _NS

    @functools.partial(
        pl.kernel,
        out_type=jax.ShapeDtypeStruct((2, _NP, 128), jnp.float32),
        mesh=_MESH,
        scratch_types=[
            pltpu.VMEM_SHARED((AR, 128), jnp.float32),
            pltpu.VMEM((_CH,), jnp.int32),
            pltpu.VMEM((_CH,), jnp.int32),
            pltpu.VMEM((_HB,), jnp.int32),
            pltpu.VMEM((_HB,), jnp.int32),
            pltpu.VMEM((_HB, 128), jnp.float32),
        ],
    )
    def k(hw_hbm, src_hbm, dst_hbm, z128_hbm,
          seg_hbm, acc_sh, src_v, dst_v, idx64_v, dst64_v, rows_v):
        c = lax.axis_index("c")
        s = lax.axis_index("s")
        w = c * _NS + s

        for m in range(_NR):
            lo = m * _RN
            pltpu.sync_copy(z128_hbm, rows_v)
            for (jo, jn) in ((0, 64), (64, 64), (128, 40)):
                pltpu.sync_copy(rows_v.at[pl.ds(0, jn)],
                                acc_sh.at[pl.ds(s * 168 + jo, jn)])
            plsc.subcore_barrier()

            def body(kk, carry):
                off = (w + kk * NW) * _CH
                pltpu.sync_copy(src_hbm.at[pl.ds(off, _CH)], src_v)
                pltpu.sync_copy(dst_hbm.at[pl.ds(off, _CH)], dst_v)
                for h in range(2):
                    _split64(src_v, idx64_v, h)
                    for q in range(4):
                        d = dst_v[pl.ds(h * 64 + q * 16, 16)]
                        t = d - lo
                        ok = (t >= 0) & (t < _RN)
                        dst64_v[pl.ds(q * 16, 16)] = jnp.where(ok, t, _TRASH)
                    pltpu.sync_copy(hw_hbm.at[idx64_v], rows_v)
                    pltpu.sync_copy(rows_v, acc_sh.at[dst64_v], add=True)
                return carry

            lax.fori_loop(0, NCH, body, 0)
            plsc.subcore_barrier()
            for (jo, jn) in ((0, 64), (64, 64), (128, 32)):
                pltpu.sync_copy(acc_sh.at[pl.ds(s * WR + jo, jn)],
                                rows_v.at[pl.ds(0, jn)])
                pltpu.sync_copy(rows_v.at[pl.ds(0, jn)],
                                seg_hbm.at[c].at[pl.ds(lo + s * WR + jo, jn)])
            plsc.subcore_barrier()

    return k(hw2, src, dst, z128)


def _sc_gather(ab, idx_all):
    NQ = _P // _CH
    NW = _NC * _NS
    NCH = NQ // NW

    @functools.partial(
        pl.kernel,
        out_type=jax.ShapeDtypeStruct((4, _P, 128), jnp.float32),
        mesh=_MESH,
        scratch_types=[
            pltpu.VMEM((_CH,), jnp.int32),
            pltpu.VMEM((_HB,), jnp.int32),
            pltpu.VMEM((_HB, 128), jnp.float32),
        ],
    )
    def k(ab_hbm, idx_hbm, g_hbm, idx_v, idx64_v, rows_v):
        c = lax.axis_index("c")
        s = lax.axis_index("s")
        w = c * _NS + s
        for t in range(4):

            def body(kk, carry):
                off = (w + kk * NW) * _CH
                pltpu.sync_copy(idx_hbm.at[t].at[pl.ds(off, _CH)], idx_v)
                for h in range(2):
                    _split64(idx_v, idx64_v, h)
                    pltpu.sync_copy(ab_hbm.at[idx64_v], rows_v)
                    pltpu.sync_copy(
                        rows_v, g_hbm.at[t].at[pl.ds(off + h * _HB, _HB)])
                return carry

            lax.fori_loop(0, NCH, body, 0)

    return k(ab, idx_all)


def _pad_edges(edge_index):
    src = jnp.pad(edge_index[0], (0, _EP - _E))
    dst = jnp.pad(edge_index[1], (0, _EP - _E), constant_values=_N)
    return src, dst


def kernel(x, edge_index0, edge_index1, pos_edge_index, neg_edge_index,
           W_self1, W_neigh1, b1, W_self2, W_neigh2, b2,
           Wp1, bp1, Wp2, bp2):
    z128 = jnp.zeros((_HB, 128), jnp.float32)
    ones128 = jnp.ones((_HB, 128), jnp.float32)

    src0, dst0 = _pad_edges(edge_index0)
    src1, dst1 = _pad_edges(edge_index1)

    xs, xw = _tc1(x, W_self1, W_neigh1)
    seg1, deg = _sc_seg1(xw, src0, dst0, z128, ones128)
    hw2, hs2 = _tc2(xs, seg1, deg[0], b1.reshape(1, 256), W_neigh2, W_self2)
    seg2 = _sc_seg2(hw2, src1, dst1, z128)
    ab = _tc3(hs2, seg2, b2.reshape(1, 64), Wp1[:64], Wp1[64:],
              bp1.reshape(1, 64))

    ne = pos_edge_index.shape[1]
    pad = _P - ne
    idx_all = jnp.concatenate([
        jnp.pad(pos_edge_index, ((0, 0), (0, pad))),
        jnp.pad(neg_edge_index, ((0, 0), (0, pad))),
    ], axis=0)
    g = _sc_gather(ab, idx_all)
    pos_s, neg_s = _tc4(g, Wp2.reshape(1, 64), bp2.reshape(1, 1))
    return pos_s[:ne, 0], neg_s[:ne, 0]


# speedup vs baseline: 1.7040x; 1.1073x over previous
import functools

import jax
import jax.numpy as jnp
from jax import lax
from jax.experimental import pallas as pl
from jax.experimental.pallas import tpu as pltpu
from jax.experimental.pallas import tpu_sc as plsc

_N = 10000
_NP = 10240
_E = 160000
_EP = 163840
_NC = 2
_NS = 16
_P = 53248


def _tc1(x, w_self1, w_neigh1):
    R = 1000

    def body(x_ref, ws_ref, wn_ref, xs_ref, xw_ref):
        xx = x_ref[...]
        xs_ref[...] = jnp.dot(xx, ws_ref[...], preferred_element_type=jnp.float32)
        xw = jnp.dot(xx, wn_ref[...], preferred_element_type=jnp.float32)
        xw_ref[0, :, :] = xw[:, :128]
        xw_ref[1, :, :] = xw[:, 128:]

    return pl.pallas_call(
        body,
        grid=(_N // R,),
        in_specs=[pl.BlockSpec((R, 512), lambda i: (i, 0)),
                  pl.BlockSpec((512, 256), lambda i: (0, 0)),
                  pl.BlockSpec((512, 256), lambda i: (0, 0))],
        out_specs=[pl.BlockSpec((R, 256), lambda i: (i, 0)),
                   pl.BlockSpec((2, R, 128), lambda i: (0, i, 0))],
        out_shape=[jax.ShapeDtypeStruct((_N, 256), jnp.float32),
                   jax.ShapeDtypeStruct((2, _N, 128), jnp.float32)],
    )(x, w_self1, w_neigh1)


def _tc2(xs, seg1, deg1, b1, w_neigh2, w_self2):
    R = 1000

    def body(xs_ref, sg_ref, dg_ref, b1_ref, wn_ref, ws_ref, hw_ref, hs_ref):
        deg = jnp.maximum(dg_ref[:, 0:1], 1.0)
        seg = jnp.concatenate([sg_ref[0], sg_ref[1]], axis=-1)
        h = jnp.maximum(xs_ref[...] + seg / deg + b1_ref[...], 0.0)
        hw = jnp.dot(h, wn_ref[...], preferred_element_type=jnp.float32)
        hw_ref[...] = jnp.concatenate(
            [hw, jnp.ones((R, 64), jnp.float32)], axis=1)
        hs_ref[...] = jnp.dot(h, ws_ref[...], preferred_element_type=jnp.float32)

    return pl.pallas_call(
        body,
        grid=(_N // R,),
        in_specs=[pl.BlockSpec((R, 256), lambda i: (i, 0)),
                  pl.BlockSpec((2, R, 128), lambda i: (0, i, 0)),
                  pl.BlockSpec((R, 128), lambda i: (i, 0)),
                  pl.BlockSpec((1, 256), lambda i: (0, 0)),
                  pl.BlockSpec((256, 64), lambda i: (0, 0)),
                  pl.BlockSpec((256, 64), lambda i: (0, 0))],
        out_specs=[pl.BlockSpec((R, 128), lambda i: (i, 0)),
                   pl.BlockSpec((R, 64), lambda i: (i, 0))],
        out_shape=[jax.ShapeDtypeStruct((_N, 128), jnp.float32),
                   jax.ShapeDtypeStruct((_N, 64), jnp.float32)],
    )(xs, seg1, deg1, b1, w_neigh2, w_self2)


def _tc3(hs2, seg2, b2, wp1a, wp1b, bp1):
    R = 1000

    def body(hs_ref, sg_ref, b2_ref, wa_ref, wb_ref, bp_ref, ab_ref):
        seg = sg_ref[0] + sg_ref[1]
        deg = jnp.maximum(seg[:, 64:65], 1.0)
        h2 = hs_ref[...] + seg[:, :64] / deg + b2_ref[...]
        a = jnp.dot(h2, wa_ref[...], preferred_element_type=jnp.float32) + bp_ref[...]
        b = jnp.dot(h2, wb_ref[...], preferred_element_type=jnp.float32)
        ab_ref[...] = jnp.concatenate([a, b], axis=1)

    return pl.pallas_call(
        body,
        grid=(_N // R,),
        in_specs=[pl.BlockSpec((R, 64), lambda i: (i, 0)),
                  pl.BlockSpec((2, R, 128), lambda i: (0, i, 0)),
                  pl.BlockSpec((1, 64), lambda i: (0, 0)),
                  pl.BlockSpec((64, 64), lambda i: (0, 0)),
                  pl.BlockSpec((64, 64), lambda i: (0, 0)),
                  pl.BlockSpec((1, 64), lambda i: (0, 0))],
        out_specs=[pl.BlockSpec((R, 128), lambda i: (i, 0))],
        out_shape=[jax.ShapeDtypeStruct((_N, 128), jnp.float32)],
    )(hs2, seg2, b2, wp1a, wp1b, bp1)[0]


def _tc4(g, w2row, bp2):
    R = 3328

    def body(g_ref, w2_ref, bp2_ref, pos_ref, neg_ref):
        zp = jnp.maximum(g_ref[0, :, :64] + g_ref[1, :, 64:], 0.0)
        zn = jnp.maximum(g_ref[2, :, :64] + g_ref[3, :, 64:], 0.0)
        pos_ref[...] = jnp.sum(zp * w2_ref[...], axis=1, keepdims=True) + bp2_ref[...]
        neg_ref[...] = jnp.sum(zn * w2_ref[...], axis=1, keepdims=True) + bp2_ref[...]

    return pl.pallas_call(
        body,
        grid=(_P // R,),
        in_specs=[pl.BlockSpec((4, R, 128), lambda i: (0, i, 0)),
                  pl.BlockSpec((1, 64), lambda i: (0, 0)),
                  pl.BlockSpec((1, 1), lambda i: (0, 0))],
        out_specs=[pl.BlockSpec((R, 1), lambda i: (i, 0)),
                   pl.BlockSpec((R, 1), lambda i: (i, 0))],
        out_shape=[jax.ShapeDtypeStruct((_P, 1), jnp.float32),
                   jax.ShapeDtypeStruct((_P, 1), jnp.float32)],
    )(g, w2row, bp2)


_MESH = plsc.VectorSubcoreMesh(core_axis_name="c", subcore_axis_name="s")

_CH = 128
_HB = 64
_RW = _NP // (_NS * _HB)


def _split64(src128_v, dst64_v, h):
    for q in range(4):
        dst64_v[pl.ds(q * 16, 16)] = src128_v[pl.ds(h * 64 + q * 16, 16)]


def _sc_seg1(xw, src0, dst0, z128, ones128):
    NQ = _EP // _CH
    NCH = NQ // _NS

    @functools.partial(
        pl.kernel,
        out_type=(jax.ShapeDtypeStruct((2, _NP, 128), jnp.float32),
                  jax.ShapeDtypeStruct((2, _NP, 128), jnp.float32)),
        mesh=_MESH,
        scratch_types=[
            pltpu.VMEM_SHARED((_NP, 128), jnp.float32),
            pltpu.VMEM((_CH,), jnp.int32),
            pltpu.VMEM((_CH,), jnp.int32),
            pltpu.VMEM((_CH,), jnp.int32),
            pltpu.VMEM((_CH,), jnp.int32),
            pltpu.VMEM((_HB,), jnp.int32),
            pltpu.VMEM((_HB,), jnp.int32),
            pltpu.VMEM((_HB, 128), jnp.float32),
            pltpu.SemaphoreType.DMA,
            pltpu.SemaphoreType.DMA,
        ],
    )
    def k(xw_hbm, src0_hbm, dst0_hbm, z128_hbm, ones_hbm,
          seg_hbm, deg_hbm,
          acc_sh, src_a, dst_a, src_b, dst_b, idx64_v, dst64_v, rows_v,
          sem_a, sem_b):
        c = lax.axis_index("c")
        s = lax.axis_index("s")

        def zero_acc():
            pltpu.sync_copy(z128_hbm, rows_v)
            for j in range(_RW):
                off = (s * _RW + j) * _HB
                pltpu.sync_copy(rows_v, acc_sh.at[pl.ds(off, _HB)])

        def write_acc(out3_hbm):
            for j in range(_RW):
                off = (s * _RW + j) * _HB
                pltpu.sync_copy(acc_sh.at[pl.ds(off, _HB)], rows_v)
                pltpu.sync_copy(rows_v, out3_hbm.at[c].at[pl.ds(off, _HB)])

        zero_acc()
        plsc.subcore_barrier()

        def start_idx(kq, sv, dv, sem):
            off = (s + kq * _NS) * _CH
            pltpu.async_copy(src0_hbm.at[pl.ds(off, _CH)], sv, sem)
            pltpu.async_copy(dst0_hbm.at[pl.ds(off, _CH)], dv, sem)

        def wait_idx(sv, dv, sem):
            pltpu.make_async_copy(src0_hbm.at[pl.ds(0, _CH)], sv, sem).wait()
            pltpu.make_async_copy(dst0_hbm.at[pl.ds(0, _CH)], dv, sem).wait()

        def process(sv, dv):
            for h in range(2):
                _split64(sv, idx64_v, h)
                _split64(dv, dst64_v, h)
                pltpu.sync_copy(xw_hbm.at[c].at[idx64_v], rows_v)
                pltpu.sync_copy(rows_v, acc_sh.at[dst64_v], add=True)

        NCH2 = NCH // 2
        start_idx(0, src_a, dst_a, sem_a)

        def body(j, carry):
            k0 = 2 * j
            start_idx(k0 + 1, src_b, dst_b, sem_b)
            wait_idx(src_a, dst_a, sem_a)
            process(src_a, dst_a)

            @pl.when(j < NCH2 - 1)
            def _():
                start_idx(k0 + 2, src_a, dst_a, sem_a)

            wait_idx(src_b, dst_b, sem_b)
            process(src_b, dst_b)
            return carry

        lax.fori_loop(0, NCH2, body, 0)
        plsc.subcore_barrier()
        write_acc(seg_hbm)
        zero_acc()
        plsc.subcore_barrier()

        pltpu.sync_copy(ones_hbm, rows_v)

        def start_d(kq, dv, sem):
            off = (s + kq * _NS) * _CH
            pltpu.async_copy(dst0_hbm.at[pl.ds(off, _CH)], dv, sem)

        def wait_d(dv, sem):
            pltpu.make_async_copy(dst0_hbm.at[pl.ds(0, _CH)], dv, sem).wait()

        def scat_ones(dv):
            for h in range(2):
                _split64(dv, dst64_v, h)
                pltpu.sync_copy(rows_v, acc_sh.at[dst64_v], add=True)

        start_d(0, dst_a, sem_a)

        def body2(j, carry):
            k0 = 2 * j
            start_d(k0 + 1, dst_b, sem_b)
            wait_d(dst_a, sem_a)
            scat_ones(dst_a)

            @pl.when(j < NCH2 - 1)
            def _():
                start_d(k0 + 2, dst_a, sem_a)

            wait_d(dst_b, sem_b)
            scat_ones(dst_b)
            return carry

        lax.fori_loop(0, NCH2, body2, 0)
        plsc.subcore_barrier()
        write_acc(deg_hbm)

    return k(xw, src0, dst0, z128, ones128)


_RN = 2560
_NR = 4
_TRASH = _RN


def _sc_seg2(hw2, src, dst, z128):
    NQ = _EP // _CH
    NW = _NC * _NS
    NCH = NQ // NW
    AR = 2688
    WR = _RN // _NS

    @functools.partial(
        pl.kernel,
        out_type=jax.ShapeDtypeStruct((2, _NP, 128), jnp.float32),
        mesh=_MESH,
        scratch_types=[
            pltpu.VMEM_SHARED((AR, 128), jnp.float32),
            pltpu.VMEM((_CH,), jnp.int32),
            pltpu.VMEM((_CH,), jnp.int32),
            pltpu.VMEM((_CH,), jnp.int32),
            pltpu.VMEM((_CH,), jnp.int32),
            pltpu.VMEM((_HB,), jnp.int32),
            pltpu.VMEM((_HB,), jnp.int32),
            pltpu.VMEM((_HB, 128), jnp.float32),
            pltpu.SemaphoreType.DMA,
            pltpu.SemaphoreType.DMA,
        ],
    )
    def k(hw_hbm, src_hbm, dst_hbm, z128_hbm,
          seg_hbm, acc_sh, src_a, dst_a, src_b, dst_b, idx64_v, dst64_v,
          rows_v, sem_a, sem_b):
        c = lax.axis_index("c")
        s = lax.axis_index("s")
        w = c * _NS + s

        def start_idx(kq, sv, dv, sem):
            off = (w + kq * NW) * _CH
            pltpu.async_copy(src_hbm.at[pl.ds(off, _CH)], sv, sem)
            pltpu.async_copy(dst_hbm.at[pl.ds(off, _CH)], dv, sem)

        def wait_idx(sv, dv, sem):
            pltpu.make_async_copy(src_hbm.at[pl.ds(0, _CH)], sv, sem).wait()
            pltpu.make_async_copy(dst_hbm.at[pl.ds(0, _CH)], dv, sem).wait()

        NCH2 = NCH // 2
        for m in range(_NR):
            lo = m * _RN
            pltpu.sync_copy(z128_hbm, rows_v)
            for (jo, jn) in ((0, 64), (64, 64), (128, 40)):
                pltpu.sync_copy(rows_v.at[pl.ds(0, jn)],
                                acc_sh.at[pl.ds(s * 168 + jo, jn)])
            plsc.subcore_barrier()

            def process(sv, dv):
                for h in range(2):
                    _split64(sv, idx64_v, h)
                    for q in range(4):
                        d = dv[pl.ds(h * 64 + q * 16, 16)]
                        t = d - lo
                        ok = (t >= 0) & (t < _RN)
                        dst64_v[pl.ds(q * 16, 16)] = jnp.where(ok, t, _TRASH)
                    pltpu.sync_copy(hw_hbm.at[idx64_v], rows_v)
                    pltpu.sync_copy(rows_v, acc_sh.at[dst64_v], add=True)

            start_idx(0, src_a, dst_a, sem_a)

            def body(j, carry):
                k0 = 2 * j
                start_idx(k0 + 1, src_b, dst_b, sem_b)
                wait_idx(src_a, dst_a, sem_a)
                process(src_a, dst_a)

                @pl.when(j < NCH2 - 1)
                def _():
                    start_idx(k0 + 2, src_a, dst_a, sem_a)

                wait_idx(src_b, dst_b, sem_b)
                process(src_b, dst_b)
                return carry

            lax.fori_loop(0, NCH2, body, 0)
            plsc.subcore_barrier()
            for (jo, jn) in ((0, 64), (64, 64), (128, 32)):
                pltpu.sync_copy(acc_sh.at[pl.ds(s * WR + jo, jn)],
                                rows_v.at[pl.ds(0, jn)])
                pltpu.sync_copy(rows_v.at[pl.ds(0, jn)],
                                seg_hbm.at[c].at[pl.ds(lo + s * WR + jo, jn)])
            plsc.subcore_barrier()

    return k(hw2, src, dst, z128)


def _sc_gather(ab, idx_all):
    NQ = _P // _CH
    NW = _NC * _NS
    NCH = NQ // NW

    @functools.partial(
        pl.kernel,
        out_type=jax.ShapeDtypeStruct((4, _P, 128), jnp.float32),
        mesh=_MESH,
        scratch_types=[
            pltpu.VMEM((_CH,), jnp.int32),
            pltpu.VMEM((_CH,), jnp.int32),
            pltpu.VMEM((_HB,), jnp.int32),
            pltpu.VMEM((_HB, 128), jnp.float32),
            pltpu.SemaphoreType.DMA,
            pltpu.SemaphoreType.DMA,
        ],
    )
    def k(ab_hbm, idx_hbm, g_hbm, idx_a, idx_b, idx64_v, rows_v,
          sem_a, sem_b):
        c = lax.axis_index("c")
        s = lax.axis_index("s")
        w = c * _NS + s
        NCH2 = NCH

        for t in range(4):

            def start_idx(kq, iv, sem):
                off = (w + kq * NW) * _CH
                pltpu.async_copy(idx_hbm.at[t].at[pl.ds(off, _CH)], iv, sem)

            def wait_idx(iv, sem):
                pltpu.make_async_copy(
                    idx_hbm.at[t].at[pl.ds(0, _CH)], iv, sem).wait()

            def process(iv, kq):
                off = (w + kq * NW) * _CH
                for h in range(2):
                    _split64(iv, idx64_v, h)
                    pltpu.sync_copy(ab_hbm.at[idx64_v], rows_v)
                    pltpu.sync_copy(
                        rows_v, g_hbm.at[t].at[pl.ds(off + h * _HB, _HB)])

            start_idx(0, idx_a, sem_a)

            def body(kk, carry):
                @pl.when(kk % 2 == 0)
                def _():
                    @pl.when(kk < NCH - 1)
                    def _():
                        start_idx(kk + 1, idx_b, sem_b)

                    wait_idx(idx_a, sem_a)
                    process(idx_a, kk)

                @pl.when(kk % 2 == 1)
                def _():
                    @pl.when(kk < NCH - 1)
                    def _():
                        start_idx(kk + 1, idx_a, sem_a)

                    wait_idx(idx_b, sem_b)
                    process(idx_b, kk)

                return carry

            lax.fori_loop(0, NCH, body, 0)

    return k(ab, idx_all)


def _pad_edges(edge_index):
    src = jnp.pad(edge_index[0], (0, _EP - _E))
    dst = jnp.pad(edge_index[1], (0, _EP - _E), constant_values=_N)
    return src, dst


def kernel(x, edge_index0, edge_index1, pos_edge_index, neg_edge_index,
           W_self1, W_neigh1, b1, W_self2, W_neigh2, b2,
           Wp1, bp1, Wp2, bp2):
    z128 = jnp.zeros((_HB, 128), jnp.float32)
    ones128 = jnp.ones((_HB, 128), jnp.float32)

    src0, dst0 = _pad_edges(edge_index0)
    src1, dst1 = _pad_edges(edge_index1)

    xs, xw = _tc1(x, W_self1, W_neigh1)
    seg1, deg = _sc_seg1(xw, src0, dst0, z128, ones128)
    hw2, hs2 = _tc2(xs, seg1, deg[0], b1.reshape(1, 256), W_neigh2, W_self2)
    seg2 = _sc_seg2(hw2, src1, dst1, z128)
    ab = _tc3(hs2, seg2, b2.reshape(1, 64), Wp1[:64], Wp1[64:],
              bp1.reshape(1, 64))

    ne = pos_edge_index.shape[1]
    pad = _P - ne
    idx_all = jnp.concatenate([
        jnp.pad(pos_edge_index, ((0, 0), (0, pad))),
        jnp.pad(neg_edge_index, ((0, 0), (0, pad))),
    ], axis=0)
    g = _sc_gather(ab, idx_all)
    pos_s, neg_s = _tc4(g, Wp2.reshape(1, 64), bp2.reshape(1, 1))
    return pos_s[:ne, 0], neg_s[:ne, 0]
